# baseline (speedup 1.0000x reference)
import jax
import jax.numpy as jnp
from jax.experimental import pallas as pl
from jax.experimental.pallas import tpu as pltpu

B, D, H, L = 128, 1024, 1024, 2
CH = 1024
NC = 4 * H // CH
NCH = L * NC
K = 3
NQ = H // CH


def _body(xt_ref, m_ref, h0_ref, c0_ref,
          wih0_ref, whh0_ref, wih1_ref, whh1_ref, b_ref,
          out_ref, nh_ref, nc_ref,
          wbuf, ig_ref, fg_ref, tc_ref, h1s_ref, sem):

    def issue(i, k):
        l = i // NC
        c = i - l * NC
        row = c * CH

        @pl.when(l == 0)
        def _():
            pltpu.make_async_copy(wih0_ref.at[pl.ds(row, CH), :],
                                  wbuf.at[k, 0], sem.at[k, 0]).start()
            pltpu.make_async_copy(whh0_ref.at[pl.ds(row, CH), :],
                                  wbuf.at[k, 1], sem.at[k, 1]).start()

        @pl.when(l == 1)
        def _():
            pltpu.make_async_copy(wih1_ref.at[pl.ds(row, CH), :],
                                  wbuf.at[k, 0], sem.at[k, 0]).start()
            pltpu.make_async_copy(whh1_ref.at[pl.ds(row, CH), :],
                                  wbuf.at[k, 1], sem.at[k, 1]).start()

    for i in range(K):
        issue(jnp.int32(i), jnp.int32(i))

    def step(i, _):
        k = jax.lax.rem(i, K)
        l = i // NC
        c = i - l * NC
        q = c // NQ
        col = (c - q * NQ) * CH
        cols = pl.ds(col, CH)

        pltpu.make_async_copy(wih0_ref.at[pl.ds(0, CH), :],
                              wbuf.at[k, 0], sem.at[k, 0]).wait()
        pltpu.make_async_copy(whh0_ref.at[pl.ds(0, CH), :],
                              wbuf.at[k, 1], sem.at[k, 1]).wait()

        h_prev = h0_ref[l]
        inp = jnp.where(l == 0, xt_ref[...], h1s_ref[...])
        gt = (jax.lax.dot_general(wbuf[k, 0], inp, (((1,), (1,)), ((), ())),
                                  preferred_element_type=jnp.float32)
              + jax.lax.dot_general(wbuf[k, 1], h_prev, (((1,), (1,)), ((), ())),
                                    preferred_element_type=jnp.float32))
        g_blk = gt.T + b_ref[l, pl.ds(q * H + col, CH)][None, :]

        m = m_ref[...] > 0

        @pl.when(q == 0)
        def _i_gate():
            ig_ref[:, cols] = jax.nn.sigmoid(g_blk)

        @pl.when(q == 1)
        def _f_gate():
            fg_ref[:, cols] = jax.nn.sigmoid(g_blk)

        @pl.when(q == 2)
        def _g_gate():
            c_new = (fg_ref[:, cols] * c0_ref[l, :, cols]
                     + ig_ref[:, cols] * jnp.tanh(g_blk))
            tc_ref[:, cols] = jnp.tanh(c_new)
            nc_ref[l, :, cols] = jnp.where(m, c_new, c0_ref[l, :, cols])

        @pl.when(q == 3)
        def _o_gate():
            h_new = jax.nn.sigmoid(g_blk) * tc_ref[:, cols]
            nh_ref[l, :, cols] = jnp.where(m, h_new, h0_ref[l, :, cols])

            @pl.when(l == 0)
            def _save_h1():
                h1s_ref[:, cols] = h_new

            @pl.when(l == 1)
            def _write_out():
                out_ref[:, cols] = jnp.where(m, h_new, jnp.zeros_like(h_new))

        @pl.when(i + K < NCH)
        def _next():
            issue(i + K, k)

        return 0

    jax.lax.fori_loop(0, NCH, step, 0)


@jax.jit
def kernel(x, mask, h0, c0, w_ih_l0, w_hh_l0, b_ih_l0, b_hh_l0,
           w_ih_l1, w_hh_l1, b_ih_l1, b_hh_l1):
    xt = x[:, 0, :]
    bias = jnp.stack([b_ih_l0 + b_hh_l0, b_ih_l1 + b_hh_l1])
    mf = (mask > 0).astype(jnp.float32)[:, None]

    vmem = pl.BlockSpec(memory_space=pltpu.MemorySpace.VMEM)
    hbm = pl.BlockSpec(memory_space=pltpu.MemorySpace.HBM)

    out, new_h, new_c = pl.pallas_call(
        _body,
        in_specs=[vmem, vmem, vmem, vmem, hbm, hbm, hbm, hbm, vmem],
        out_specs=[vmem, vmem, vmem],
        out_shape=[
            jax.ShapeDtypeStruct((B, H), jnp.float32),
            jax.ShapeDtypeStruct((L, B, H), jnp.float32),
            jax.ShapeDtypeStruct((L, B, H), jnp.float32),
        ],
        scratch_shapes=[
            pltpu.VMEM((K, 2, CH, H), jnp.float32),
            pltpu.VMEM((B, H), jnp.float32),
            pltpu.VMEM((B, H), jnp.float32),
            pltpu.VMEM((B, H), jnp.float32),
            pltpu.VMEM((B, H), jnp.float32),
            pltpu.SemaphoreType.DMA((K, 2)),
        ],
    )(xt, mf, h0, c0, w_ih_l0, w_hh_l0, w_ih_l1, w_hh_l1, bias)

    return out[:, None, :], new_h, new_c


# speedup vs baseline: 1.0309x; 1.0309x over previous
import jax
import jax.numpy as jnp
from jax.experimental import pallas as pl
from jax.experimental.pallas import tpu as pltpu

B, D, H, L = 128, 1024, 1024, 2
CH = 512
NC = 4 * H // CH
NCH = L * NC
K = 4
NQ = H // CH


def _body(xt_ref, m_ref, h0_ref, c0_ref,
          wih0_ref, whh0_ref, wih1_ref, whh1_ref, b_ref,
          out_ref, nh_ref, nc_ref,
          wbuf, ig_ref, fg_ref, tc_ref, h1s_ref,
          nh_s, nc_s, out_s, sem, osem):

    def issue(i, k):
        l = i // NC
        c = i - l * NC
        row = c * CH

        @pl.when(l == 0)
        def _():
            pltpu.make_async_copy(wih0_ref.at[pl.ds(row, CH), :],
                                  wbuf.at[k, 0], sem.at[k, 0]).start()
            pltpu.make_async_copy(whh0_ref.at[pl.ds(row, CH), :],
                                  wbuf.at[k, 1], sem.at[k, 1]).start()

        @pl.when(l == 1)
        def _():
            pltpu.make_async_copy(wih1_ref.at[pl.ds(row, CH), :],
                                  wbuf.at[k, 0], sem.at[k, 0]).start()
            pltpu.make_async_copy(whh1_ref.at[pl.ds(row, CH), :],
                                  wbuf.at[k, 1], sem.at[k, 1]).start()

    for i in range(K):
        issue(jnp.int32(i), jnp.int32(i))

    def step(i, _):
        k = jax.lax.rem(i, K)
        l = i // NC
        c = i - l * NC
        q = c // NQ
        sub = c - q * NQ
        col = sub * CH
        cols = pl.ds(col, CH)

        pltpu.make_async_copy(wih0_ref.at[pl.ds(0, CH), :],
                              wbuf.at[k, 0], sem.at[k, 0]).wait()
        pltpu.make_async_copy(whh0_ref.at[pl.ds(0, CH), :],
                              wbuf.at[k, 1], sem.at[k, 1]).wait()

        h_prev = h0_ref[l]
        inp = jnp.where(l == 0, xt_ref[...], h1s_ref[...])
        gt = (jax.lax.dot_general(wbuf[k, 0], inp, (((1,), (1,)), ((), ())),
                                  preferred_element_type=jnp.float32)
              + jax.lax.dot_general(wbuf[k, 1], h_prev, (((1,), (1,)), ((), ())),
                                    preferred_element_type=jnp.float32))
        g_blk = gt.T + b_ref[l, pl.ds(q * H + col, CH)][None, :]

        m = m_ref[...] > 0

        @pl.when(q == 0)
        def _i_gate():
            ig_ref[:, cols] = jax.nn.sigmoid(g_blk)

        @pl.when(q == 1)
        def _f_gate():
            fg_ref[:, cols] = jax.nn.sigmoid(g_blk)

        @pl.when(q == 2)
        def _g_gate():
            c_new = (fg_ref[:, cols] * c0_ref[l, :, cols]
                     + ig_ref[:, cols] * jnp.tanh(g_blk))
            tc_ref[:, cols] = jnp.tanh(c_new)
            nc_s[l, :, cols] = jnp.where(m, c_new, c0_ref[l, :, cols])
            pltpu.make_async_copy(nc_s.at[l, :, cols], nc_ref.at[l, :, cols],
                                  osem.at[0]).start()

        @pl.when(q == 3)
        def _o_gate():
            h_new = jax.nn.sigmoid(g_blk) * tc_ref[:, cols]
            nh_s[l, :, cols] = jnp.where(m, h_new, h0_ref[l, :, cols])
            pltpu.make_async_copy(nh_s.at[l, :, cols], nh_ref.at[l, :, cols],
                                  osem.at[1]).start()

            @pl.when(l == 0)
            def _save_h1():
                h1s_ref[:, cols] = h_new

            @pl.when(l == 1)
            def _write_out():
                out_s[:, cols] = jnp.where(m, h_new, jnp.zeros_like(h_new))
                pltpu.make_async_copy(out_s.at[:, cols], out_ref.at[:, cols],
                                      osem.at[2]).start()

        @pl.when(i + K < NCH)
        def _next():
            issue(i + K, k)

        return 0

    jax.lax.fori_loop(0, NCH, step, 0)

    for l in range(L):
        for sub in range(NQ):
            cols = pl.ds(sub * CH, CH)
            pltpu.make_async_copy(nc_s.at[l, :, cols], nc_ref.at[l, :, cols],
                                  osem.at[0]).wait()
            pltpu.make_async_copy(nh_s.at[l, :, cols], nh_ref.at[l, :, cols],
                                  osem.at[1]).wait()
    for sub in range(NQ):
        cols = pl.ds(sub * CH, CH)
        pltpu.make_async_copy(out_s.at[:, cols], out_ref.at[:, cols],
                              osem.at[2]).wait()


@jax.jit
def kernel(x, mask, h0, c0, w_ih_l0, w_hh_l0, b_ih_l0, b_hh_l0,
           w_ih_l1, w_hh_l1, b_ih_l1, b_hh_l1):
    xt = x[:, 0, :]
    bias = jnp.stack([b_ih_l0 + b_hh_l0, b_ih_l1 + b_hh_l1])
    mf = (mask > 0).astype(jnp.float32)[:, None]

    vmem = pl.BlockSpec(memory_space=pltpu.MemorySpace.VMEM)
    hbm = pl.BlockSpec(memory_space=pltpu.MemorySpace.HBM)

    out, new_h, new_c = pl.pallas_call(
        _body,
        in_specs=[vmem, vmem, vmem, vmem, hbm, hbm, hbm, hbm, vmem],
        out_specs=[hbm, hbm, hbm],
        out_shape=[
            jax.ShapeDtypeStruct((B, H), jnp.float32),
            jax.ShapeDtypeStruct((L, B, H), jnp.float32),
            jax.ShapeDtypeStruct((L, B, H), jnp.float32),
        ],
        scratch_shapes=[
            pltpu.VMEM((K, 2, CH, H), jnp.float32),
            pltpu.VMEM((B, H), jnp.float32),
            pltpu.VMEM((B, H), jnp.float32),
            pltpu.VMEM((B, H), jnp.float32),
            pltpu.VMEM((B, H), jnp.float32),
            pltpu.VMEM((L, B, H), jnp.float32),
            pltpu.VMEM((L, B, H), jnp.float32),
            pltpu.VMEM((B, H), jnp.float32),
            pltpu.SemaphoreType.DMA((K, 2)),
            pltpu.SemaphoreType.DMA((3,)),
        ],
    )(xt, mf, h0, c0, w_ih_l0, w_hh_l0, w_ih_l1, w_hh_l1, bias)

    return out[:, None, :], new_h, new_c
